# baseline (speedup 1.0000x reference)
import functools

import jax
import jax.numpy as jnp
from jax import lax
from jax.experimental import pallas as pl
from jax.experimental.pallas import tpu as pltpu
from jax.experimental.pallas import tpu_sc as plsc

N_NODES = 10000
D = 128
LN_EPS = 1e-5

NC = 2
NS = 16
NW = NC * NS
K = 128
NB = 80
E_PAD = NW * NB * K
CH = 8

ACC_ROWS = 10240
RPT = ACC_ROWS // NS

_STRIPE_CHUNKS = [(i * K, K) for i in range(RPT // K)]
if RPT % K:
  _STRIPE_CHUNKS.append(((RPT // K) * K, RPT % K))

_MESH = plsc.VectorSubcoreMesh(core_axis_name="c", subcore_axis_name="s")


@functools.partial(
    pl.kernel,
    out_type=jax.ShapeDtypeStruct((NC, ACC_ROWS), jnp.float32),
    mesh=_MESH,
    compiler_params=pltpu.CompilerParams(needs_layout_passes=False),
    scratch_types=[
        pltpu.VMEM((NB, K), jnp.int32),
        pltpu.VMEM((ACC_ROWS,), jnp.float32),
        pltpu.VMEM((NS, RPT), jnp.float32),
        pltpu.VMEM((RPT,), jnp.float32),
        pltpu.VMEM_SHARED((NS, ACC_ROWS), jnp.float32),
        pltpu.SemaphoreType.DMA,
    ],
)
def _hist_call(dst_hbm, deg_hbm, dst_v, hist_v, col_v, row_v, hist_sh, sem):
  c = lax.axis_index("c")
  s = lax.axis_index("s")
  wid = c * NS + s
  pltpu.sync_copy(dst_hbm.at[wid], dst_v)

  zeros = jnp.zeros((16,), jnp.float32)

  def zbody(i, carry):
    hist_v[pl.ds(i * 16, 16)] = zeros
    return carry

  lax.fori_loop(0, ACC_ROWS // 16, zbody, 0)

  ones = jnp.ones((16,), jnp.float32)

  def body(r, carry):
    for l in range(K // 16):
      idx = dst_v[r, pl.ds(l * 16, 16)]
      plsc.addupdate_scatter(hist_v, [idx], ones)
    return carry

  lax.fori_loop(0, NB, body, 0)
  pltpu.sync_copy(hist_v, hist_sh.at[s])
  plsc.subcore_barrier()
  base = s * RPT
  pltpu.sync_copy(hist_sh.at[:, pl.ds(base, RPT)], col_v)

  def rbody(j, carry):
    acc = col_v[0, pl.ds(j * 16, 16)]
    for t in range(1, NS):
      acc = acc + col_v[t, pl.ds(j * 16, 16)]
    row_v[pl.ds(j * 16, 16)] = acc
    return carry

  lax.fori_loop(0, RPT // 16, rbody, 0)
  pltpu.sync_copy(row_v, deg_hbm.at[c].at[pl.ds(base, RPT)])


@functools.partial(
    pl.kernel,
    out_type=jax.ShapeDtypeStruct((NC, ACC_ROWS, D), jnp.float32),
    mesh=_MESH,
    scratch_types=[
        pltpu.VMEM((CH, K), jnp.int32),
        pltpu.VMEM((CH, K), jnp.int32),
        pltpu.VMEM((K, D), jnp.float32),
        pltpu.VMEM((K, D), jnp.float32),
        pltpu.VMEM_SHARED((ACC_ROWS, D), jnp.float32),
        pltpu.SemaphoreType.DMA,
        pltpu.SemaphoreType.DMA,
    ],
)
def _agg_call(src_hbm, dst_hbm, y_hbm, zeros_hbm, acc_hbm, src_v, dst_v, buf0,
              buf1, acc_sh, sem0, sem1):
  c = lax.axis_index("c")
  s = lax.axis_index("s")
  wid = c * NS + s
  pltpu.sync_copy(zeros_hbm, buf0)
  base = s * RPT
  for off, sz in _STRIPE_CHUNKS:
    pltpu.sync_copy(buf0.at[pl.ds(0, sz)], acc_sh.at[pl.ds(base + off, sz)])
  plsc.subcore_barrier()

  def chunk_body(ch, carry):
    pltpu.sync_copy(src_hbm.at[wid].at[pl.ds(ch * CH, CH)], src_v)
    pltpu.sync_copy(dst_hbm.at[wid].at[pl.ds(ch * CH, CH)], dst_v)
    pltpu.async_copy(y_hbm.at[src_v.at[0]], buf0, sem0)

    def body(i, c2):
      j = 2 * i
      pltpu.async_copy(y_hbm.at[src_v.at[j + 1]], buf1, sem1)
      pltpu.make_async_copy(y_hbm.at[src_v.at[j]], buf0, sem0).wait()
      pltpu.sync_copy(buf0, acc_sh.at[dst_v.at[j]], add=True)

      @pl.when(j + 2 < CH)
      def _():
        pltpu.async_copy(y_hbm.at[src_v.at[j + 2]], buf0, sem0)

      pltpu.make_async_copy(y_hbm.at[src_v.at[j + 1]], buf1, sem1).wait()
      pltpu.sync_copy(buf1, acc_sh.at[dst_v.at[j + 1]], add=True)
      return c2

    lax.fori_loop(0, CH // 2, body, 0)
    return carry

  lax.fori_loop(0, NB // CH, chunk_body, 0)
  plsc.subcore_barrier()
  for off, sz in _STRIPE_CHUNKS:
    pltpu.sync_copy(acc_sh.at[pl.ds(base + off, sz)],
                    acc_hbm.at[c].at[pl.ds(base + off, sz)])


_ROWS_BLK = 1000
_GRID = N_NODES // _ROWS_BLK


def _y_body(d_ref0, d_ref1, x_ref, w_ref, y_ref):
  deg = d_ref0[0] + d_ref1[0] + 1.0
  dinv = lax.rsqrt(deg)
  xw = jnp.dot(x_ref[...], w_ref[...], preferred_element_type=jnp.float32)
  y_ref[...] = xw * dinv


def _y_call(d0, d1, x, W):
  return pl.pallas_call(
      _y_body,
      grid=(_GRID,),
      in_specs=[
          pl.BlockSpec((1, _ROWS_BLK, 1), lambda i: (i, 0, 0)),
          pl.BlockSpec((1, _ROWS_BLK, 1), lambda i: (i, 0, 0)),
          pl.BlockSpec((_ROWS_BLK, D), lambda i: (i, 0)),
          pl.BlockSpec((D, D), lambda i: (0, 0)),
      ],
      out_specs=pl.BlockSpec((_ROWS_BLK, D), lambda i: (i, 0)),
      out_shape=jax.ShapeDtypeStruct((N_NODES, D), jnp.float32),
  )(d0, d1, x, W)


def _fin_body(a_ref0, a_ref1, y_ref, d_ref0, d_ref1, b_ref, g_ref, be_ref,
              o_ref):
  deg = d_ref0[0] + d_ref1[0] + 1.0
  dinv = lax.rsqrt(deg)
  srow = (a_ref0[0] + a_ref1[0] + y_ref[...]) * dinv + b_ref[...]
  mean = jnp.mean(srow, axis=1, keepdims=True)
  cen = srow - mean
  var = jnp.mean(cen * cen, axis=1, keepdims=True)
  o_ref[...] = jnp.maximum(
      cen * lax.rsqrt(var + LN_EPS) * g_ref[...] + be_ref[...], 0.0)


def _fin_call(acc, y, d0, d1, b, gamma, beta):
  return pl.pallas_call(
      _fin_body,
      grid=(_GRID,),
      in_specs=[
          pl.BlockSpec((1, _ROWS_BLK, D), lambda i: (0, i, 0)),
          pl.BlockSpec((1, _ROWS_BLK, D), lambda i: (1, i, 0)),
          pl.BlockSpec((_ROWS_BLK, D), lambda i: (i, 0)),
          pl.BlockSpec((1, _ROWS_BLK, 1), lambda i: (i, 0, 0)),
          pl.BlockSpec((1, _ROWS_BLK, 1), lambda i: (i, 0, 0)),
          pl.BlockSpec((1, D), lambda i: (0, 0)),
          pl.BlockSpec((1, D), lambda i: (0, 0)),
          pl.BlockSpec((1, D), lambda i: (0, 0)),
      ],
      out_specs=pl.BlockSpec((_ROWS_BLK, D), lambda i: (i, 0)),
      out_shape=jax.ShapeDtypeStruct((N_NODES, D), jnp.float32),
  )(acc, acc, y, d0, d1, b, gamma, beta)


def kernel(x, edge_index, W, b, gamma, beta):
  e = edge_index.shape[1]
  pad = E_PAD - e
  src = jnp.concatenate(
      [edge_index[0], jnp.zeros((pad,), jnp.int32)]).reshape(NW, NB, K)
  dst = jnp.concatenate(
      [edge_index[1],
       jnp.full((pad,), N_NODES, jnp.int32)]).reshape(NW, NB, K)

  zeros_rows = jnp.zeros((K, D), jnp.float32)

  deg_p = _hist_call(dst)
  d0 = deg_p[0, :N_NODES].reshape(_GRID, _ROWS_BLK, 1)
  d1 = deg_p[1, :N_NODES].reshape(_GRID, _ROWS_BLK, 1)
  y = _y_call(d0, d1, x, W)
  acc = _agg_call(src, dst, y, zeros_rows)
  return _fin_call(acc, y, d0, d1, b.reshape(1, D), gamma.reshape(1, D),
                   beta.reshape(1, D))


# speedup vs baseline: 1.0098x; 1.0098x over previous
import functools

import jax
import jax.numpy as jnp
from jax import lax
from jax.experimental import pallas as pl
from jax.experimental.pallas import tpu as pltpu
from jax.experimental.pallas import tpu_sc as plsc

N_NODES = 10000
D = 128
LN_EPS = 1e-5

NC = 2
NS = 16
NW = NC * NS
K = 128
NB = 80
E_PAD = NW * NB * K
CH = 8

ACC_ROWS = 10240
RPT = ACC_ROWS // NS

_STRIPE_CHUNKS = [(i * K, K) for i in range(RPT // K)]
if RPT % K:
  _STRIPE_CHUNKS.append(((RPT // K) * K, RPT % K))

_MESH = plsc.VectorSubcoreMesh(core_axis_name="c", subcore_axis_name="s")


@functools.partial(
    pl.kernel,
    out_type=jax.ShapeDtypeStruct((NC, ACC_ROWS), jnp.float32),
    mesh=_MESH,
    compiler_params=pltpu.CompilerParams(needs_layout_passes=False),
    scratch_types=[
        pltpu.VMEM((NB, K), jnp.int32),
        pltpu.VMEM((ACC_ROWS,), jnp.float32),
        pltpu.VMEM((NS, RPT), jnp.float32),
        pltpu.VMEM((RPT,), jnp.float32),
        pltpu.VMEM_SHARED((NS, ACC_ROWS), jnp.float32),
        pltpu.SemaphoreType.DMA,
    ],
)
def _hist_call(dst_hbm, deg_hbm, dst_v, hist_v, col_v, row_v, hist_sh, sem):
  c = lax.axis_index("c")
  s = lax.axis_index("s")
  wid = c * NS + s
  pltpu.sync_copy(dst_hbm.at[wid], dst_v)

  zeros = jnp.zeros((16,), jnp.float32)

  def zbody(i, carry):
    hist_v[pl.ds(i * 16, 16)] = zeros
    return carry

  lax.fori_loop(0, ACC_ROWS // 16, zbody, 0)

  ones = jnp.ones((16,), jnp.float32)

  def body(r, carry):
    for l in range(K // 16):
      idx = dst_v[r, pl.ds(l * 16, 16)]
      plsc.addupdate_scatter(hist_v, [idx], ones)
    return carry

  lax.fori_loop(0, NB, body, 0)
  pltpu.sync_copy(hist_v, hist_sh.at[s])
  plsc.subcore_barrier()
  base = s * RPT
  pltpu.sync_copy(hist_sh.at[:, pl.ds(base, RPT)], col_v)

  def rbody(j, carry):
    acc = col_v[0, pl.ds(j * 16, 16)]
    for t in range(1, NS):
      acc = acc + col_v[t, pl.ds(j * 16, 16)]
    row_v[pl.ds(j * 16, 16)] = acc
    return carry

  lax.fori_loop(0, RPT // 16, rbody, 0)
  pltpu.sync_copy(row_v, deg_hbm.at[c].at[pl.ds(base, RPT)])


@functools.partial(
    pl.kernel,
    out_type=jax.ShapeDtypeStruct((NC, ACC_ROWS, D), jnp.float32),
    mesh=_MESH,
    scratch_types=[
        pltpu.VMEM((CH, K), jnp.int32),
        pltpu.VMEM((CH, K), jnp.int32),
        pltpu.VMEM((K, D), jnp.float32),
        pltpu.VMEM((K, D), jnp.float32),
        pltpu.VMEM_SHARED((ACC_ROWS, D), jnp.float32),
        pltpu.SemaphoreType.DMA,
        pltpu.SemaphoreType.DMA,
    ],
)
def _agg_call(src_hbm, dst_hbm, y_hbm, zeros_hbm, acc_hbm, src_v, dst_v, buf0,
              buf1, acc_sh, sem0, sem1):
  c = lax.axis_index("c")
  s = lax.axis_index("s")
  wid = c * NS + s
  pltpu.sync_copy(zeros_hbm, buf0)
  base = s * RPT
  for off, sz in _STRIPE_CHUNKS:
    pltpu.sync_copy(buf0.at[pl.ds(0, sz)], acc_sh.at[pl.ds(base + off, sz)])
  plsc.subcore_barrier()

  def chunk_body(ch, carry):
    pltpu.sync_copy(src_hbm.at[wid].at[pl.ds(ch * CH, CH)], src_v)
    pltpu.sync_copy(dst_hbm.at[wid].at[pl.ds(ch * CH, CH)], dst_v)
    pltpu.async_copy(y_hbm.at[src_v.at[0]], buf0, sem0)

    def body(i, c2):
      j = 2 * i
      pltpu.async_copy(y_hbm.at[src_v.at[j + 1]], buf1, sem1)
      pltpu.make_async_copy(y_hbm.at[src_v.at[j]], buf0, sem0).wait()
      pltpu.sync_copy(buf0, acc_sh.at[dst_v.at[j]], add=True)

      @pl.when(j + 2 < CH)
      def _():
        pltpu.async_copy(y_hbm.at[src_v.at[j + 2]], buf0, sem0)

      pltpu.make_async_copy(y_hbm.at[src_v.at[j + 1]], buf1, sem1).wait()
      pltpu.sync_copy(buf1, acc_sh.at[dst_v.at[j + 1]], add=True)
      return c2

    lax.fori_loop(0, CH // 2, body, 0)
    return carry

  lax.fori_loop(0, NB // CH, chunk_body, 0)
  plsc.subcore_barrier()
  for off, sz in _STRIPE_CHUNKS:
    pltpu.sync_copy(acc_sh.at[pl.ds(base + off, sz)],
                    acc_hbm.at[c].at[pl.ds(base + off, sz)])


_ROWS_BLK = 1000
_GRID = N_NODES // _ROWS_BLK


def _y_body(d_ref0, d_ref1, x_ref, w_ref, y_ref):
  deg = d_ref0[0] + d_ref1[0] + 1.0
  dinv = lax.rsqrt(deg)
  xw = jnp.dot(x_ref[...], w_ref[...], preferred_element_type=jnp.float32)
  y_ref[...] = xw * dinv


def _y_call(d0, d1, x, W):
  return pl.pallas_call(
      _y_body,
      grid=(_GRID,),
      in_specs=[
          pl.BlockSpec((1, _ROWS_BLK, 1), lambda i: (i, 0, 0)),
          pl.BlockSpec((1, _ROWS_BLK, 1), lambda i: (i, 0, 0)),
          pl.BlockSpec((_ROWS_BLK, D), lambda i: (i, 0)),
          pl.BlockSpec((D, D), lambda i: (0, 0)),
      ],
      out_specs=pl.BlockSpec((_ROWS_BLK, D), lambda i: (i, 0)),
      out_shape=jax.ShapeDtypeStruct((N_NODES, D), jnp.float32),
  )(d0, d1, x, W)


def _fin_body(a_ref0, a_ref1, y_ref, d_ref0, d_ref1, b_ref, g_ref, be_ref,
              o_ref):
  deg = d_ref0[0] + d_ref1[0] + 1.0
  dinv = lax.rsqrt(deg)
  srow = (a_ref0[0] + a_ref1[0] + y_ref[...]) * dinv + b_ref[...]
  mean = jnp.mean(srow, axis=1, keepdims=True)
  cen = srow - mean
  var = jnp.mean(cen * cen, axis=1, keepdims=True)
  o_ref[...] = jnp.maximum(
      cen * lax.rsqrt(var + LN_EPS) * g_ref[...] + be_ref[...], 0.0)


def _fin_call(acc, y, d0, d1, b, gamma, beta):
  return pl.pallas_call(
      _fin_body,
      grid=(_GRID,),
      in_specs=[
          pl.BlockSpec((1, _ROWS_BLK, D), lambda i: (0, i, 0)),
          pl.BlockSpec((1, _ROWS_BLK, D), lambda i: (1, i, 0)),
          pl.BlockSpec((_ROWS_BLK, D), lambda i: (i, 0)),
          pl.BlockSpec((1, _ROWS_BLK, 1), lambda i: (i, 0, 0)),
          pl.BlockSpec((1, _ROWS_BLK, 1), lambda i: (i, 0, 0)),
          pl.BlockSpec((1, D), lambda i: (0, 0)),
          pl.BlockSpec((1, D), lambda i: (0, 0)),
          pl.BlockSpec((1, D), lambda i: (0, 0)),
      ],
      out_specs=pl.BlockSpec((_ROWS_BLK, D), lambda i: (i, 0)),
      out_shape=jax.ShapeDtypeStruct((N_NODES, D), jnp.float32),
  )(acc, acc, y, d0, d1, b, gamma, beta)


def kernel(x, edge_index, W, b, gamma, beta):
  e = edge_index.shape[1]
  pad = E_PAD - e
  src = jnp.concatenate(
      [edge_index[0], jnp.zeros((pad,), jnp.int32)]).reshape(NW, NB, K)
  pad_dst = N_NODES + jnp.arange(pad, dtype=jnp.int32) % (ACC_ROWS - N_NODES)
  dst = jnp.concatenate([edge_index[1], pad_dst]).reshape(NW, NB, K)

  zeros_rows = jnp.zeros((K, D), jnp.float32)

  deg_p = _hist_call(dst)
  d0 = deg_p[0, :N_NODES].reshape(_GRID, _ROWS_BLK, 1)
  d1 = deg_p[1, :N_NODES].reshape(_GRID, _ROWS_BLK, 1)
  y = _y_call(d0, d1, x, W)
  acc = _agg_call(src, dst, y, zeros_rows)
  return _fin_call(acc, y, d0, d1, b.reshape(1, D), gamma.reshape(1, D),
                   beta.reshape(1, D))


# speedup vs baseline: 3.1038x; 3.0736x over previous
import functools

import jax
import jax.numpy as jnp
from jax import lax
from jax.experimental import pallas as pl
from jax.experimental.pallas import tpu as pltpu
from jax.experimental.pallas import tpu_sc as plsc

N_NODES = 10000
D = 128
LN_EPS = 1e-5

NC = 2
NS = 16
NW = NC * NS
K = 128
NB = 80
E_PAD = NW * NB * K
CH = 8

ACC_ROWS = 10240
RPT = ACC_ROWS // NS

_STRIPE_CHUNKS = [(i * K, K) for i in range(RPT // K)]
if RPT % K:
  _STRIPE_CHUNKS.append(((RPT // K) * K, RPT % K))

_MESH = plsc.VectorSubcoreMesh(core_axis_name="c", subcore_axis_name="s")


@functools.partial(
    pl.kernel,
    out_type=jax.ShapeDtypeStruct((NC, ACC_ROWS), jnp.float32),
    mesh=_MESH,
    compiler_params=pltpu.CompilerParams(needs_layout_passes=False),
    scratch_types=[
        pltpu.VMEM((NB, K), jnp.int32),
        pltpu.VMEM((ACC_ROWS,), jnp.float32),
        pltpu.VMEM((NS, RPT), jnp.float32),
        pltpu.VMEM((RPT,), jnp.float32),
        pltpu.VMEM_SHARED((NS, ACC_ROWS), jnp.float32),
        pltpu.SemaphoreType.DMA,
    ],
)
def _hist_call(dst_hbm, deg_hbm, dst_v, hist_v, col_v, row_v, hist_sh, sem):
  c = lax.axis_index("c")
  s = lax.axis_index("s")
  wid = c * NS + s
  pltpu.sync_copy(dst_hbm.at[wid], dst_v)

  zeros = jnp.zeros((16,), jnp.float32)

  def zbody(i, carry):
    hist_v[pl.ds(i * 16, 16)] = zeros
    return carry

  lax.fori_loop(0, ACC_ROWS // 16, zbody, 0)

  ones = jnp.ones((16,), jnp.float32)

  def body(r, carry):
    for l in range(K // 16):
      idx = dst_v[r, pl.ds(l * 16, 16)]
      plsc.addupdate_scatter(hist_v, [idx], ones)
    return carry

  lax.fori_loop(0, NB, body, 0)
  pltpu.sync_copy(hist_v, hist_sh.at[s])
  plsc.subcore_barrier()
  base = s * RPT
  pltpu.sync_copy(hist_sh.at[:, pl.ds(base, RPT)], col_v)

  def rbody(j, carry):
    acc = col_v[0, pl.ds(j * 16, 16)]
    for t in range(1, NS):
      acc = acc + col_v[t, pl.ds(j * 16, 16)]
    row_v[pl.ds(j * 16, 16)] = acc
    return carry

  lax.fori_loop(0, RPT // 16, rbody, 0)
  pltpu.sync_copy(row_v, deg_hbm.at[c].at[pl.ds(base, RPT)])


@functools.partial(
    pl.kernel,
    out_type=jax.ShapeDtypeStruct((NC, ACC_ROWS, D), jnp.float32),
    mesh=_MESH,
    scratch_types=[
        pltpu.VMEM((CH, K), jnp.int32),
        pltpu.VMEM((CH, K), jnp.int32),
        pltpu.VMEM((K, D), jnp.float32),
        pltpu.VMEM((K, D), jnp.float32),
        pltpu.VMEM_SHARED((ACC_ROWS, D), jnp.float32),
        pltpu.SemaphoreType.DMA,
        pltpu.SemaphoreType.DMA,
    ],
)
def _agg_call(src_hbm, dst_hbm, y_hbm, zeros_hbm, acc_hbm, src_v, dst_v, buf0,
              buf1, acc_sh, sem0, sem1):
  c = lax.axis_index("c")
  s = lax.axis_index("s")
  wid = c * NS + s
  pltpu.sync_copy(zeros_hbm, buf0)
  base = s * RPT
  for off, sz in _STRIPE_CHUNKS:
    pltpu.sync_copy(buf0.at[pl.ds(0, sz)], acc_sh.at[pl.ds(base + off, sz)])
  plsc.subcore_barrier()

  def chunk_body(ch, carry):
    pltpu.sync_copy(src_hbm.at[wid].at[pl.ds(ch * CH, CH)], src_v)
    pltpu.sync_copy(dst_hbm.at[wid].at[pl.ds(ch * CH, CH)], dst_v)
    pltpu.async_copy(y_hbm.at[src_v.at[0]], buf0, sem0)

    def body(i, c2):
      j = 2 * i
      pltpu.async_copy(y_hbm.at[src_v.at[j + 1]], buf1, sem1)
      pltpu.make_async_copy(y_hbm.at[src_v.at[j]], buf0, sem0).wait()
      pltpu.sync_copy(buf0, acc_sh.at[dst_v.at[j]], add=True)

      @pl.when(j + 2 < CH)
      def _():
        pltpu.async_copy(y_hbm.at[src_v.at[j + 2]], buf0, sem0)

      pltpu.make_async_copy(y_hbm.at[src_v.at[j + 1]], buf1, sem1).wait()
      pltpu.sync_copy(buf1, acc_sh.at[dst_v.at[j + 1]], add=True)
      return c2

    lax.fori_loop(0, CH // 2, body, 0)
    return carry

  lax.fori_loop(0, NB // CH, chunk_body, 0)
  plsc.subcore_barrier()
  for off, sz in _STRIPE_CHUNKS:
    pltpu.sync_copy(acc_sh.at[pl.ds(base + off, sz)],
                    acc_hbm.at[c].at[pl.ds(base + off, sz)])


_ROWS_BLK = 1000
_GRID = N_NODES // _ROWS_BLK


def _y_body(d_ref0, d_ref1, x_ref, w_ref, y_ref):
  deg = d_ref0[0] + d_ref1[0] + 1.0
  dinv = lax.rsqrt(deg)
  xw = jnp.dot(x_ref[...], w_ref[...], preferred_element_type=jnp.float32)
  y_ref[...] = xw * dinv


def _y_call(d0, d1, x, W):
  return pl.pallas_call(
      _y_body,
      grid=(_GRID,),
      in_specs=[
          pl.BlockSpec((1, _ROWS_BLK, 1), lambda i: (i, 0, 0)),
          pl.BlockSpec((1, _ROWS_BLK, 1), lambda i: (i, 0, 0)),
          pl.BlockSpec((_ROWS_BLK, D), lambda i: (i, 0)),
          pl.BlockSpec((D, D), lambda i: (0, 0)),
      ],
      out_specs=pl.BlockSpec((_ROWS_BLK, D), lambda i: (i, 0)),
      out_shape=jax.ShapeDtypeStruct((N_NODES, D), jnp.float32),
  )(d0, d1, x, W)


def _fin_body(a_ref0, a_ref1, y_ref, d_ref0, d_ref1, b_ref, g_ref, be_ref,
              o_ref):
  deg = d_ref0[0] + d_ref1[0] + 1.0
  dinv = lax.rsqrt(deg)
  srow = (a_ref0[0] + a_ref1[0] + y_ref[...]) * dinv + b_ref[...]
  mean = jnp.mean(srow, axis=1, keepdims=True)
  cen = srow - mean
  var = jnp.mean(cen * cen, axis=1, keepdims=True)
  o_ref[...] = jnp.maximum(
      cen * lax.rsqrt(var + LN_EPS) * g_ref[...] + be_ref[...], 0.0)


def _fin_call(acc, y, d0, d1, b, gamma, beta):
  return pl.pallas_call(
      _fin_body,
      grid=(_GRID,),
      in_specs=[
          pl.BlockSpec((1, _ROWS_BLK, D), lambda i: (0, i, 0)),
          pl.BlockSpec((1, _ROWS_BLK, D), lambda i: (1, i, 0)),
          pl.BlockSpec((_ROWS_BLK, D), lambda i: (i, 0)),
          pl.BlockSpec((1, _ROWS_BLK, 1), lambda i: (i, 0, 0)),
          pl.BlockSpec((1, _ROWS_BLK, 1), lambda i: (i, 0, 0)),
          pl.BlockSpec((1, D), lambda i: (0, 0)),
          pl.BlockSpec((1, D), lambda i: (0, 0)),
          pl.BlockSpec((1, D), lambda i: (0, 0)),
      ],
      out_specs=pl.BlockSpec((_ROWS_BLK, D), lambda i: (i, 0)),
      out_shape=jax.ShapeDtypeStruct((N_NODES, D), jnp.float32),
  )(acc, acc, y, d0, d1, b, gamma, beta)


def kernel(x, edge_index, W, b, gamma, beta):
  e = edge_index.shape[1]
  pad = E_PAD - e
  pad_src = jnp.arange(pad, dtype=jnp.int32) % N_NODES
  pad_dst = N_NODES + jnp.arange(pad, dtype=jnp.int32) % (ACC_ROWS - N_NODES)
  src = jnp.concatenate([edge_index[0], pad_src]).reshape(NW, NB, K)
  dst = jnp.concatenate([edge_index[1], pad_dst]).reshape(NW, NB, K)

  zeros_rows = jnp.zeros((K, D), jnp.float32)

  deg_p = _hist_call(dst)
  d0 = deg_p[0, :N_NODES].reshape(_GRID, _ROWS_BLK, 1)
  d1 = deg_p[1, :N_NODES].reshape(_GRID, _ROWS_BLK, 1)
  y = _y_call(d0, d1, x, W)
  acc = _agg_call(src, dst, y, zeros_rows)
  return _fin_call(acc, y, d0, d1, b.reshape(1, D), gamma.reshape(1, D),
                   beta.reshape(1, D))
